# Initial kernel scaffold; baseline (speedup 1.0000x reference)
import jax
import jax.numpy as jnp
from jax.experimental import pallas as pl


def kernel(x, edge_index, batch, W1_rel, b1_rel, W1_root, W2_rel, b2_rel, W2_root, W3_rel, b3_rel, W3_root, Wh, bh, Wo, bo):
    raise NotImplementedError("write your pallas kernel here")


# speedup vs baseline: 3.0418x; 3.0418x over previous
import functools

import jax
import jax.numpy as jnp
from jax import lax
from jax.experimental import pallas as pl
from jax.experimental.pallas import tpu as pltpu, tpu_sc as plsc

N = 10000
E = 320000
D = 128
G = 64

PAD_N = 10240
NC = 2
NS = 16
CB = 128
NCH = 80
IB = 16
NBLK = NCH // IB
EPW = NCH * CB
EPAD = NC * NS * EPW
RPT = PAD_N // NS

_sc_mesh = plsc.VectorSubcoreMesh(core_axis_name="c", subcore_axis_name="s")


@functools.partial(
    pl.kernel,
    out_type=jax.ShapeDtypeStruct((NC * PAD_N, D), jnp.float32),
    mesh=_sc_mesh,
    scratch_types=(
        pltpu.VMEM((IB, CB), jnp.int32),
        pltpu.VMEM((IB, CB), jnp.int32),
        pltpu.VMEM((CB, D), jnp.float32),
        pltpu.VMEM((CB, D), jnp.float32),
        pltpu.VMEM_SHARED((PAD_N, D), jnp.float32),
        pltpu.SemaphoreType.DMA,
        pltpu.SemaphoreType.DMA,
    ),
)
def _sc_agg(x_hbm, src_hbm, dst_hbm, out_hbm,
            src_v, dst_v, buf0, buf1, acc, sem0, sem1):
    c = lax.axis_index("c")
    s = lax.axis_index("s")
    wid = c * NS + s

    def _zrow(i, carry):
        for k in range(D // 16):
            buf0[i, pl.ds(k * 16, 16)] = jnp.zeros((16,), jnp.float32)
        return carry
    lax.fori_loop(0, CB, _zrow, 0)
    for k in range(RPT // CB):
        pltpu.sync_copy(buf0, acc.at[pl.ds(s * RPT + k * CB, CB)])
    plsc.subcore_barrier()

    def _block(blk, carry):
        row0 = wid * NCH + blk * IB
        pltpu.sync_copy(src_hbm.at[pl.ds(row0, IB)], src_v)
        pltpu.sync_copy(dst_hbm.at[pl.ds(row0, IB)], dst_v)
        pltpu.async_copy(x_hbm.at[src_v.at[0]], buf0, sem0)

        def _body(j2, carry2):
            j = 2 * j2
            pltpu.async_copy(x_hbm.at[src_v.at[j + 1]], buf1, sem1)
            pltpu.make_async_copy(x_hbm.at[src_v.at[j]], buf0, sem0).wait()
            pltpu.sync_copy(buf0, acc.at[dst_v.at[j]], add=True)

            @pl.when(j2 < IB // 2 - 1)
            def _():
                pltpu.async_copy(x_hbm.at[src_v.at[j + 2]], buf0, sem0)

            pltpu.make_async_copy(x_hbm.at[src_v.at[j + 1]], buf1, sem1).wait()
            pltpu.sync_copy(buf1, acc.at[dst_v.at[j + 1]], add=True)
            return carry2

        lax.fori_loop(0, IB // 2, _body, 0)
        return carry

    lax.fori_loop(0, NBLK, _block, 0)
    plsc.subcore_barrier()

    for k in range(RPT // CB):
        r0 = s * RPT + k * CB
        pltpu.sync_copy(acc.at[pl.ds(r0, CB)], buf0)
        pltpu.sync_copy(buf0, out_hbm.at[pl.ds(c * PAD_N + r0, CB)])


BN = 640
N_BLK = PAD_N // BN


def _dense_body(p0_ref, p1_ref, h_ref, wr_ref, br_ref, wt_ref, o_ref):
    agg = p0_ref[...] + p1_ref[...]
    o = jnp.dot(agg, wr_ref[...], preferred_element_type=jnp.float32)
    o += jnp.dot(h_ref[...], wt_ref[...], preferred_element_type=jnp.float32)
    o_ref[...] = jnp.maximum(o + br_ref[...], 0.0)


def _dense(parts, h, wr, br, wt):
    return pl.pallas_call(
        _dense_body,
        grid=(N_BLK,),
        in_specs=[
            pl.BlockSpec((BN, D), lambda i: (i, 0)),
            pl.BlockSpec((BN, D), lambda i: (i + N_BLK, 0)),
            pl.BlockSpec((BN, D), lambda i: (i, 0)),
            pl.BlockSpec((D, D), lambda i: (0, 0)),
            pl.BlockSpec((1, D), lambda i: (0, 0)),
            pl.BlockSpec((D, D), lambda i: (0, 0)),
        ],
        out_specs=pl.BlockSpec((BN, D), lambda i: (i, 0)),
        out_shape=jax.ShapeDtypeStruct((PAD_N, D), jnp.float32),
    )(parts, parts, h, wr, br, wt)


PB = 1000
P_BLK = N // PB


def _pool_body(b_ref, h_ref, wh_ref, bh_ref, wo_ref, bo_ref, o_ref, g_acc):
    i = pl.program_id(0)

    @pl.when(i == 0)
    def _():
        g_acc[...] = jnp.zeros_like(g_acc)

    bb = b_ref[0, 0, :]
    oh = (bb[:, None] == lax.broadcasted_iota(jnp.int32, (PB, G), 1))
    g_acc[...] += lax.dot_general(
        oh.astype(jnp.float32), h_ref[...],
        (((0,), (0,)), ((), ())), preferred_element_type=jnp.float32)

    @pl.when(i == P_BLK - 1)
    def _():
        g = jnp.maximum(
            jnp.dot(g_acc[...], wh_ref[...],
                    preferred_element_type=jnp.float32) + bh_ref[...], 0.0)
        o_ref[...] = jnp.dot(
            g, wo_ref[...], preferred_element_type=jnp.float32) + bo_ref[...]


def _pool_mlp(batch3, h, wh, bh, wo, bo):
    return pl.pallas_call(
        _pool_body,
        grid=(P_BLK,),
        in_specs=[
            pl.BlockSpec((1, 1, PB), lambda i: (i, 0, 0)),
            pl.BlockSpec((PB, D), lambda i: (i, 0)),
            pl.BlockSpec((D, D), lambda i: (0, 0)),
            pl.BlockSpec((1, D), lambda i: (0, 0)),
            pl.BlockSpec((D, D), lambda i: (0, 0)),
            pl.BlockSpec((1, D), lambda i: (0, 0)),
        ],
        out_specs=pl.BlockSpec((G, D), lambda i: (0, 0)),
        out_shape=jax.ShapeDtypeStruct((G, D), jnp.float32),
        scratch_shapes=[pltpu.VMEM((G, D), jnp.float32)],
    )(batch3, h, wh, bh, wo, bo)


@jax.jit
def kernel(x, edge_index, batch, W1_rel, b1_rel, W1_root, W2_rel, b2_rel,
           W2_root, W3_rel, b3_rel, W3_root, Wh, bh, Wo, bo):
    x = x.astype(jnp.float32)
    src = edge_index[0].astype(jnp.int32)
    dst = edge_index[1].astype(jnp.int32)
    batch = batch.astype(jnp.int32)

    pad = jnp.full((EPAD - E,), N, jnp.int32)
    src_p = jnp.concatenate([src, pad]).reshape(NC * NS * NCH, CB)
    dst_p = jnp.concatenate([dst, pad]).reshape(NC * NS * NCH, CB)

    h = jnp.zeros((PAD_N, D), jnp.float32).at[:N].set(x)

    for wr, br, wt in ((W1_rel, b1_rel, W1_root),
                       (W2_rel, b2_rel, W2_root),
                       (W3_rel, b3_rel, W3_root)):
        parts = _sc_agg(h, src_p, dst_p)
        h = _dense(parts, h, wr, br.reshape(1, D), wt)

    batch3 = batch.reshape(P_BLK, 1, PB)
    return _pool_mlp(batch3, h[:N], Wh, bh.reshape(1, D),
                     Wo, bo.reshape(1, D))


# speedup vs baseline: 10.5292x; 3.4615x over previous
import functools

import jax
import jax.numpy as jnp
from jax import lax
from jax.experimental import pallas as pl
from jax.experimental.pallas import tpu as pltpu, tpu_sc as plsc

N = 10000
E = 320000
D = 128
G = 64

PAD_N = 10240
NC = 2
NS = 16
CB = 128
NCH = 80
IB = 16
NBLK = NCH // IB
EPW = NCH * CB
EPAD = NC * NS * EPW
RPT = PAD_N // NS

_sc_mesh = plsc.VectorSubcoreMesh(core_axis_name="c", subcore_axis_name="s")


@functools.partial(
    pl.kernel,
    out_type=jax.ShapeDtypeStruct((NC * PAD_N, D), jnp.float32),
    mesh=_sc_mesh,
    scratch_types=(
        pltpu.VMEM((IB, CB), jnp.int32),
        pltpu.VMEM((IB, CB), jnp.int32),
        pltpu.VMEM((CB, D), jnp.float32),
        pltpu.VMEM((CB, D), jnp.float32),
        pltpu.VMEM_SHARED((PAD_N, D), jnp.float32),
        pltpu.SemaphoreType.DMA,
        pltpu.SemaphoreType.DMA,
    ),
)
def _sc_agg(x_hbm, src_hbm, dst_hbm, out_hbm,
            src_v, dst_v, buf0, buf1, acc, sem0, sem1):
    c = lax.axis_index("c")
    s = lax.axis_index("s")
    wid = c * NS + s

    def _zrow(i, carry):
        for k in range(D // 16):
            buf0[i, pl.ds(k * 16, 16)] = jnp.zeros((16,), jnp.float32)
        return carry
    lax.fori_loop(0, CB, _zrow, 0)
    for k in range(RPT // CB):
        pltpu.sync_copy(buf0, acc.at[pl.ds(s * RPT + k * CB, CB)])
    plsc.subcore_barrier()

    def _block(blk, carry):
        row0 = wid * NCH + blk * IB
        pltpu.sync_copy(src_hbm.at[pl.ds(row0, IB)], src_v)
        pltpu.sync_copy(dst_hbm.at[pl.ds(row0, IB)], dst_v)
        pltpu.async_copy(x_hbm.at[src_v.at[0]], buf0, sem0)

        def _body(j2, carry2):
            j = 2 * j2
            pltpu.async_copy(x_hbm.at[src_v.at[j + 1]], buf1, sem1)
            pltpu.make_async_copy(x_hbm.at[src_v.at[j]], buf0, sem0).wait()
            pltpu.sync_copy(buf0, acc.at[dst_v.at[j]], add=True)

            @pl.when(j2 < IB // 2 - 1)
            def _():
                pltpu.async_copy(x_hbm.at[src_v.at[j + 2]], buf0, sem0)

            pltpu.make_async_copy(x_hbm.at[src_v.at[j + 1]], buf1, sem1).wait()
            pltpu.sync_copy(buf1, acc.at[dst_v.at[j + 1]], add=True)
            return carry2

        lax.fori_loop(0, IB // 2, _body, 0)
        return carry

    lax.fori_loop(0, NBLK, _block, 0)
    plsc.subcore_barrier()

    for k in range(RPT // CB):
        r0 = s * RPT + k * CB
        pltpu.sync_copy(acc.at[pl.ds(r0, CB)], buf0)
        pltpu.sync_copy(buf0, out_hbm.at[pl.ds(c * PAD_N + r0, CB)])


BN = 640
N_BLK = PAD_N // BN


def _dense_body(p0_ref, p1_ref, h_ref, wr_ref, br_ref, wt_ref, o_ref):
    agg = p0_ref[...] + p1_ref[...]
    o = jnp.dot(agg, wr_ref[...], preferred_element_type=jnp.float32)
    o += jnp.dot(h_ref[...], wt_ref[...], preferred_element_type=jnp.float32)
    o_ref[...] = jnp.maximum(o + br_ref[...], 0.0)


def _dense(parts, h, wr, br, wt):
    return pl.pallas_call(
        _dense_body,
        grid=(N_BLK,),
        in_specs=[
            pl.BlockSpec((BN, D), lambda i: (i, 0)),
            pl.BlockSpec((BN, D), lambda i: (i + N_BLK, 0)),
            pl.BlockSpec((BN, D), lambda i: (i, 0)),
            pl.BlockSpec((D, D), lambda i: (0, 0)),
            pl.BlockSpec((1, D), lambda i: (0, 0)),
            pl.BlockSpec((D, D), lambda i: (0, 0)),
        ],
        out_specs=pl.BlockSpec((BN, D), lambda i: (i, 0)),
        out_shape=jax.ShapeDtypeStruct((PAD_N, D), jnp.float32),
    )(parts, parts, h, wr, br, wt)


PB = 1000
P_BLK = N // PB


def _pool_body(b_ref, h_ref, wh_ref, bh_ref, wo_ref, bo_ref, o_ref, g_acc):
    i = pl.program_id(0)

    @pl.when(i == 0)
    def _():
        g_acc[...] = jnp.zeros_like(g_acc)

    bb = b_ref[0, 0, :]
    oh = (bb[:, None] == lax.broadcasted_iota(jnp.int32, (PB, G), 1))
    g_acc[...] += lax.dot_general(
        oh.astype(jnp.float32), h_ref[...],
        (((0,), (0,)), ((), ())), preferred_element_type=jnp.float32)

    @pl.when(i == P_BLK - 1)
    def _():
        g = jnp.maximum(
            jnp.dot(g_acc[...], wh_ref[...],
                    preferred_element_type=jnp.float32) + bh_ref[...], 0.0)
        o_ref[...] = jnp.dot(
            g, wo_ref[...], preferred_element_type=jnp.float32) + bo_ref[...]


def _pool_mlp(batch3, h, wh, bh, wo, bo):
    return pl.pallas_call(
        _pool_body,
        grid=(P_BLK,),
        in_specs=[
            pl.BlockSpec((1, 1, PB), lambda i: (i, 0, 0)),
            pl.BlockSpec((PB, D), lambda i: (i, 0)),
            pl.BlockSpec((D, D), lambda i: (0, 0)),
            pl.BlockSpec((1, D), lambda i: (0, 0)),
            pl.BlockSpec((D, D), lambda i: (0, 0)),
            pl.BlockSpec((1, D), lambda i: (0, 0)),
        ],
        out_specs=pl.BlockSpec((G, D), lambda i: (0, 0)),
        out_shape=jax.ShapeDtypeStruct((G, D), jnp.float32),
        scratch_shapes=[pltpu.VMEM((G, D), jnp.float32)],
    )(batch3, h, wh, bh, wo, bo)


@jax.jit
def kernel(x, edge_index, batch, W1_rel, b1_rel, W1_root, W2_rel, b2_rel,
           W2_root, W3_rel, b3_rel, W3_root, Wh, bh, Wo, bo):
    x = x.astype(jnp.float32)
    src = edge_index[0].astype(jnp.int32)
    dst = edge_index[1].astype(jnp.int32)
    batch = batch.astype(jnp.int32)

    pad = N + (jnp.arange(EPAD - E, dtype=jnp.int32) % (PAD_N - N))
    src_p = jnp.concatenate([src, pad]).reshape(NC * NS * NCH, CB)
    dst_p = jnp.concatenate([dst, pad]).reshape(NC * NS * NCH, CB)

    h = jnp.zeros((PAD_N, D), jnp.float32).at[:N].set(x)

    for wr, br, wt in ((W1_rel, b1_rel, W1_root),
                       (W2_rel, b2_rel, W2_root),
                       (W3_rel, b3_rel, W3_root)):
        parts = _sc_agg(h, src_p, dst_p)
        h = _dense(parts, h, wr, br.reshape(1, D), wt)

    batch3 = batch.reshape(P_BLK, 1, PB)
    return _pool_mlp(batch3, h, Wh, bh.reshape(1, D),
                     Wo, bo.reshape(1, D))
